# baseline (speedup 1.0000x reference)
import functools
import math

import jax
import jax.numpy as jnp
from jax import lax
from jax.experimental import pallas as pl
from jax.experimental.pallas import tpu as pltpu
from jax.experimental.pallas import tpu_sc as plsc

D_MODEL = 64
SCALE = math.sqrt(D_MODEL)
NC = 2
NS = 16
NW = NC * NS
L = 16
G = 128


def _emb_body(idx_hbm, table_hbm, out_hbm, idx_v, rows_v, gsem):
    wid = lax.axis_index("s") * NC + lax.axis_index("c")
    b_per_w = idx_hbm.shape[0] // NW
    ngroups = b_per_w // G
    base = wid * b_per_w

    pltpu.sync_copy(idx_hbm.at[pl.ds(base, b_per_w)], idx_v)

    def group_body(g, carry):
        off = base + g * G
        pltpu.async_copy(
            table_hbm.at[idx_v.at[pl.ds(g * G, G)]], rows_v, gsem
        ).wait()

        def scale_row(i, c):
            for j in range(D_MODEL // L):
                rows_v[i, pl.ds(j * L, L)] = rows_v[i, pl.ds(j * L, L)] * SCALE
            return c
        lax.fori_loop(0, G, scale_row, 0)

        pltpu.sync_copy(rows_v, out_hbm.at[pl.ds(off, G)])
        return carry

    lax.fori_loop(0, ngroups, group_body, 0)


def kernel(x, table):
    B = x.shape[0] * x.shape[1]
    xf = x.reshape(B).astype(jnp.int32)
    mesh = plsc.VectorSubcoreMesh(core_axis_name="c", subcore_axis_name="s")
    b_per_w = B // NW
    run = functools.partial(
        pl.kernel,
        mesh=mesh,
        out_type=jax.ShapeDtypeStruct((B, D_MODEL), jnp.float32),
        scratch_types=[
            pltpu.VMEM((b_per_w,), jnp.int32),
            pltpu.VMEM((G, D_MODEL), jnp.float32),
            pltpu.SemaphoreType.DMA,
        ],
        compiler_params=pltpu.CompilerParams(use_tc_tiling_on_sc=False),
    )(_emb_body)
    out = run(xf, table)
    return out.reshape(x.shape[0], x.shape[1], D_MODEL)


# speedup vs baseline: 1.2105x; 1.2105x over previous
import functools
import math

import jax
import jax.numpy as jnp
from jax import lax
from jax.experimental import pallas as pl
from jax.experimental.pallas import tpu as pltpu
from jax.experimental.pallas import tpu_sc as plsc

D_MODEL = 64
SCALE = math.sqrt(D_MODEL)
NC = 2
NS = 16
NW = NC * NS
L = 16
G = 128
NBUF = 4
ROW_UNROLL = 4


def _emb_body(idx_hbm, table_hbm, out_hbm, idx_v, gbufs, obufs, gsems, osems):
    wid = lax.axis_index("s") * NC + lax.axis_index("c")
    b_per_w = idx_hbm.shape[0] // NW
    ngroups = b_per_w // G
    base = wid * b_per_w

    pltpu.sync_copy(idx_hbm.at[pl.ds(base, b_per_w)], idx_v)

    def gstart(g, b):
        pltpu.make_async_copy(
            table_hbm.at[idx_v.at[pl.ds(g * G, G)]], gbufs[b], gsems[b]
        ).start()

    def gwait(b):
        pltpu.make_async_copy(
            table_hbm.at[idx_v.at[pl.ds(0, G)]], gbufs[b], gsems[b]
        ).wait()

    def ostart(g, b):
        pltpu.make_async_copy(
            obufs[b], out_hbm.at[pl.ds(base + g * G, G)], osems[b]
        ).start()

    def owait(b):
        pltpu.make_async_copy(
            obufs[b], out_hbm.at[pl.ds(base, G)], osems[b]
        ).wait()

    def scale(b):
        def rows(i, c):
            for r in range(ROW_UNROLL):
                row = i * ROW_UNROLL + r
                for j in range(D_MODEL // L):
                    sl = pl.ds(j * L, L)
                    obufs[b][row, sl] = gbufs[b][row, sl] * SCALE
            return c
        lax.fori_loop(0, G // ROW_UNROLL, rows, 0)

    for b in range(NBUF):
        gstart(b, b)

    for b in range(NBUF):
        gwait(b)
        scale(b)
        ostart(b, b)
        gstart(b + NBUF, b)

    def outer(o, c):
        for b in range(NBUF):
            g = o * NBUF + b
            gwait(b)
            owait(b)
            scale(b)
            ostart(g, b)
            gstart(g + NBUF, b)
        return c

    lax.fori_loop(1, ngroups // NBUF - 1, outer, 0)

    for b in range(NBUF):
        g = ngroups - NBUF + b
        gwait(b)
        owait(b)
        scale(b)
        ostart(g, b)

    for b in range(NBUF):
        owait(b)


def kernel(x, table):
    B = x.shape[0] * x.shape[1]
    xf = x.reshape(B).astype(jnp.int32)
    mesh = plsc.VectorSubcoreMesh(core_axis_name="c", subcore_axis_name="s")
    b_per_w = B // NW
    assert B % NW == 0 and b_per_w % (G * NBUF) == 0
    run = functools.partial(
        pl.kernel,
        mesh=mesh,
        out_type=jax.ShapeDtypeStruct((B, D_MODEL), jnp.float32),
        scratch_types=[
            pltpu.VMEM((b_per_w,), jnp.int32),
            [pltpu.VMEM((G, D_MODEL), jnp.float32) for _ in range(NBUF)],
            [pltpu.VMEM((G, D_MODEL), jnp.float32) for _ in range(NBUF)],
            [pltpu.SemaphoreType.DMA for _ in range(NBUF)],
            [pltpu.SemaphoreType.DMA for _ in range(NBUF)],
        ],
        compiler_params=pltpu.CompilerParams(use_tc_tiling_on_sc=False),
    )(_emb_body)
    out = run(xf, table)
    return out.reshape(x.shape[0], x.shape[1], D_MODEL)
